# baseline (speedup 1.0000x reference)
import functools

import jax
import jax.numpy as jnp
from jax import lax
from jax.experimental import pallas as pl
from jax.experimental.pallas import tpu as pltpu
from jax.experimental.pallas import tpu_sc as plsc

_B = 64
_B_SC = 32
_N = 8192
_L = 16
_NV = _N // _L
_U = 16
_NC = 2
_NS = 16
_ROWS_PER_W = _B_SC // (_NC * _NS)
_NEG = -1e30

_mesh = plsc.VectorSubcoreMesh(core_axis_name="c", subcore_axis_name="s")


@functools.partial(
    pl.kernel,
    out_type=jax.ShapeDtypeStruct((_B_SC, _N), jnp.float32),
    mesh=_mesh,
    scratch_types=[
        pltpu.VMEM((_N,), jnp.float32),
        pltpu.VMEM((_N + _L,), jnp.float32),
        pltpu.SemaphoreType.DMA,
        pltpu.SemaphoreType.DMA,
    ],
    compiler_params=pltpu.CompilerParams(needs_layout_passes=False),
)
def _sparsemax_sc(x_hbm, out_hbm, row_v0, act_v, sin0, sout0):
    wid = lax.axis_index("s") * _NC + lax.axis_index("c")
    row0 = wid * _ROWS_PER_W

    row_bufs = [row_v0]
    in_cps = [pltpu.async_copy(x_hbm.at[row0], row_v0, sin0)]
    out_sems = [sout0]
    out_cps = []

    for r in range(_ROWS_PER_W):
        row_v = row_bufs[r]
        in_cps[r].wait()

        def max_body(i, accs):
            base = i * (_L * _U)
            return tuple(
                jnp.maximum(a, row_v[pl.ds(base + j * _L, _L)])
                for j, a in enumerate(accs)
            )

        accs = lax.fori_loop(
            0, _NV // _U, max_body,
            tuple(jnp.full((_L,), _NEG, jnp.float32) for _ in range(_U)),
        )
        m16 = accs[0]
        for a in accs[1:]:
            m16 = jnp.maximum(m16, a)
        m = jnp.max(m16)
        lo0 = m - 1.0

        def compact_body(i, n):
            base = i * (_L * _U)
            vs = [row_v[pl.ds(base + j * _L, _L)] for j in range(_U)]
            msks = [v > lo0 for v in vs]
            cnts = [plsc.all_reduce_population_count(k)[0] for k in msks]
            for v, msk, cnt in zip(vs, msks, cnts):
                plsc.store_compressed(act_v.at[pl.ds(n, _L)], v, mask=msk)
                n = n + cnt
            return n

        n_act = lax.fori_loop(0, _NV // _U, compact_body, jnp.int32(0))
        act_v[pl.ds(n_act, _L)] = jnp.full((_L,), _NEG, jnp.float32)
        nv = (n_act + _L - 1) // _L

        def mic_cond(st):
            changed, it, _, _ = st
            return changed & (it < jnp.int32(512))

        def mic_body(st):
            _, it, k_old, tau_old = st

            def scan_body(i, c):
                kk, ss = c
                v = act_v[pl.ds(i * _L, _L)]
                msk = v > tau_old
                kk = kk + plsc.all_reduce_population_count(msk)
                ss = ss + jnp.where(msk, v, 0.0)
                return kk, ss

            k_v, s_v = lax.fori_loop(
                0, nv, scan_body,
                (jnp.zeros((_L,), jnp.int32), jnp.zeros((_L,), jnp.float32)),
            )
            tau = lax.broadcast(jnp.sum(s_v) - 1.0, (_L,)) / k_v.astype(jnp.float32)
            changed = jnp.any(k_v != k_old)
            return changed, it + 1, k_v, tau

        _, _, _, tau = lax.while_loop(
            mic_cond, mic_body,
            (jnp.bool_(True), jnp.int32(0), jnp.zeros((_L,), jnp.int32),
             lax.broadcast(lo0, (_L,))),
        )

        def out_body(i, carry):
            base = i * (_L * _U)
            for j in range(_U):
                sl = pl.ds(base + j * _L, _L)
                row_v[sl] = jnp.maximum(row_v[sl] - tau, 0.0)
            return carry

        lax.fori_loop(0, _NV // _U, out_body, jnp.int32(0))
        out_cps.append(pltpu.async_copy(row_v, out_hbm.at[row0 + r], out_sems[r]))

    for cp in out_cps:
        cp.wait()


def _tc_body(x_ref, o_ref):
    x = x_ref[...]
    m = jnp.max(x, axis=1, keepdims=True)

    def cond(st):
        changed, it, _, _ = st
        return changed & (it < jnp.int32(512))

    def body(st):
        _, it, k_old, tau = st
        msk = x > tau
        k = jnp.sum(jnp.where(msk, 1.0, 0.0), axis=1, keepdims=True)
        s = jnp.sum(jnp.where(msk, x, 0.0), axis=1, keepdims=True)
        tau_n = (s - 1.0) / k
        changed = jnp.any(k != k_old)
        return changed, it + 1, k, tau_n

    _, _, _, tau = lax.while_loop(
        cond, body,
        (jnp.bool_(True), jnp.int32(0),
         jnp.zeros((x.shape[0], 1), jnp.float32), m - 1.0),
    )
    o_ref[...] = jnp.maximum(x - tau, 0.0)


_sparsemax_tc = pl.pallas_call(
    _tc_body,
    grid=(1,),
    in_specs=[pl.BlockSpec((_B - _B_SC, _N), lambda i: (1, 0))],
    out_specs=pl.BlockSpec((_B - _B_SC, _N), lambda i: (1, 0)),
    out_shape=jax.ShapeDtypeStruct((_B, _N), jnp.float32),
)


def _cat_body(full_ref, top_ref, o_ref, sem):
    del full_ref
    pltpu.async_copy(top_ref, o_ref.at[pl.ds(0, _B_SC)], sem).wait()


_assemble = pl.pallas_call(
    _cat_body,
    in_specs=[
        pl.BlockSpec(memory_space=pl.ANY),
        pl.BlockSpec(memory_space=pl.ANY),
    ],
    out_specs=pl.BlockSpec(memory_space=pl.ANY),
    out_shape=jax.ShapeDtypeStruct((_B, _N), jnp.float32),
    scratch_shapes=[pltpu.SemaphoreType.DMA],
    input_output_aliases={0: 0},
)


def kernel(input):
    top = _sparsemax_sc(input)
    bot = _sparsemax_tc(input)
    return _assemble(bot, top)


# speedup vs baseline: 2.2471x; 2.2471x over previous
import functools

import jax
import jax.numpy as jnp
from jax import lax
from jax.experimental import pallas as pl
from jax.experimental.pallas import tpu as pltpu
from jax.experimental.pallas import tpu_sc as plsc

_B = 64
_N = 8192
_L = 16
_NV = _N // _L
_U = 16
_NC = 2
_NS = 16
_ROWS_PER_W = _B // (_NC * _NS)
_NEG = -1e30

_mesh = plsc.VectorSubcoreMesh(core_axis_name="c", subcore_axis_name="s")


@functools.partial(
    pl.kernel,
    out_type=jax.ShapeDtypeStruct((_B, _N), jnp.float32),
    mesh=_mesh,
    scratch_types=[
        pltpu.VMEM((_N,), jnp.float32),
        pltpu.VMEM((_N,), jnp.float32),
        pltpu.VMEM((_N + _L,), jnp.float32),
        pltpu.SemaphoreType.DMA,
        pltpu.SemaphoreType.DMA,
        pltpu.SemaphoreType.DMA,
        pltpu.SemaphoreType.DMA,
    ],
    compiler_params=pltpu.CompilerParams(needs_layout_passes=False),
)
def _sparsemax_sc(x_hbm, out_hbm, row_v0, row_v1, act_v, sin0, sin1, sout0, sout1):
    wid = lax.axis_index("s") * _NC + lax.axis_index("c")
    row0 = wid * _ROWS_PER_W

    row_bufs = [row_v0, row_v1]
    in_cps = [
        pltpu.async_copy(x_hbm.at[row0 + r], row_bufs[r], sem)
        for r, sem in ((0, sin0), (1, sin1))
    ]
    out_sems = [sout0, sout1]
    out_cps = []

    for r in range(_ROWS_PER_W):
        row_v = row_bufs[r]
        in_cps[r].wait()

        def max_body(i, accs):
            base = i * (_L * _U)
            return tuple(
                jnp.maximum(a, row_v[pl.ds(base + j * _L, _L)])
                for j, a in enumerate(accs)
            )

        accs = lax.fori_loop(
            0, _NV // _U, max_body,
            tuple(jnp.full((_L,), _NEG, jnp.float32) for _ in range(_U)),
        )
        m16 = accs[0]
        for a in accs[1:]:
            m16 = jnp.maximum(m16, a)
        m = jnp.max(m16)
        lo0 = m - 1.0

        def compact_body(i, n):
            base = i * (_L * _U)
            vs = [row_v[pl.ds(base + j * _L, _L)] for j in range(_U)]
            msks = [v > lo0 for v in vs]
            cnts = [plsc.all_reduce_population_count(k)[0] for k in msks]
            for v, msk, cnt in zip(vs, msks, cnts):
                plsc.store_compressed(act_v.at[pl.ds(n, _L)], v, mask=msk)
                n = n + cnt
            return n

        n_act = lax.fori_loop(0, _NV // _U, compact_body, jnp.int32(0))
        act_v[pl.ds(n_act, _L)] = jnp.full((_L,), _NEG, jnp.float32)
        nv = (n_act + _L - 1) // _L

        def mic_cond(st):
            changed, it, _, _ = st
            return changed & (it < jnp.int32(512))

        def mic_body(st):
            _, it, k_old, tau_old = st

            def scan_body(i, c):
                kk, ss = c
                v = act_v[pl.ds(i * _L, _L)]
                msk = v > tau_old
                kk = kk + plsc.all_reduce_population_count(msk)
                ss = ss + jnp.where(msk, v, 0.0)
                return kk, ss

            k_v, s_v = lax.fori_loop(
                0, nv, scan_body,
                (jnp.zeros((_L,), jnp.int32), jnp.zeros((_L,), jnp.float32)),
            )
            tau = lax.broadcast(jnp.sum(s_v) - 1.0, (_L,)) / k_v.astype(jnp.float32)
            changed = jnp.any(k_v != k_old)
            return changed, it + 1, k_v, tau

        _, _, _, tau = lax.while_loop(
            mic_cond, mic_body,
            (jnp.bool_(True), jnp.int32(0), jnp.zeros((_L,), jnp.int32),
             lax.broadcast(lo0, (_L,))),
        )

        def out_body(i, carry):
            base = i * (_L * _U)
            for j in range(_U):
                sl = pl.ds(base + j * _L, _L)
                row_v[sl] = jnp.maximum(row_v[sl] - tau, 0.0)
            return carry

        lax.fori_loop(0, _NV // _U, out_body, jnp.int32(0))
        out_cps.append(pltpu.async_copy(row_v, out_hbm.at[row0 + r], out_sems[r]))

    for cp in out_cps:
        cp.wait()


def kernel(input):
    return _sparsemax_sc(input)
